# baseline (speedup 1.0000x reference)
import functools

import jax
import jax.numpy as jnp
from jax import lax
from jax.experimental import pallas as pl
from jax.experimental.pallas import tpu as pltpu
from jax.experimental.pallas import tpu_sc as plsc

N_NODES = 10000
N_PAD = 10240
N_EDGES = 320000
D = 128
LANES = 16

NC = 2
NS = 16
NW = NC * NS
E_PER_W = N_EDGES // NW
CHUNK = 80
N_CHUNKS = E_PER_W // CHUNK
ROWS_PER_TILE = N_PAD // NS
WROWS = CHUNK

K = 4
N_PIPE = N_CHUNKS - 1
N_GROUPS = N_PIPE // K

_mesh = plsc.VectorSubcoreMesh(core_axis_name="c", subcore_axis_name="s")


@functools.partial(
    pl.kernel,
    out_type=jax.ShapeDtypeStruct((NC, N_PAD, D), jnp.float32),
    mesh=_mesh,
    scratch_types=[
        [pltpu.VMEM((CHUNK,), jnp.int32) for _ in range(K)],
        [pltpu.VMEM((CHUNK,), jnp.int32) for _ in range(K)],
        [pltpu.VMEM((CHUNK, D), jnp.float32) for _ in range(K)],
        pltpu.VMEM_SHARED((N_PAD, D), jnp.float32),
        pltpu.SemaphoreType.DMA((K,)),
        pltpu.SemaphoreType.DMA((K,)),
        pltpu.SemaphoreType.DMA((K,)),
    ],
)
def _mp_sc(x_hbm, src_hbm, dst_hbm, out_hbm, sbufs, dbufs, rbufs,
           acc_sh, isem, gsem, ssem):
    cid = lax.axis_index("c")
    sid = lax.axis_index("s")
    wid = sid * NC + cid
    e_base = wid * E_PER_W

    zeros = jnp.zeros((LANES,), jnp.float32)

    def _zero_row(i, _):
        for c in range(D // LANES):
            rbufs[3][i, pl.ds(c * LANES, LANES)] = zeros
        return 0

    def _zero_acc(j, _):
        pltpu.sync_copy(rbufs[3],
                        acc_sh.at[pl.ds(sid * ROWS_PER_TILE + j * WROWS, WROWS)])
        return 0

    def _fetch_idx(c, b):
        base = e_base + c * CHUNK
        pltpu.async_copy(src_hbm.at[pl.ds(base, CHUNK)], sbufs[b], isem.at[b])
        pltpu.async_copy(dst_hbm.at[pl.ds(base, CHUNK)], dbufs[b], isem.at[b])

    def _wait_idx(c, b):
        base = e_base + c * CHUNK
        pltpu.make_async_copy(src_hbm.at[pl.ds(base, CHUNK)], sbufs[b],
                              isem.at[b]).wait()
        pltpu.make_async_copy(dst_hbm.at[pl.ds(base, CHUNK)], dbufs[b],
                              isem.at[b]).wait()

    def _wait_scatter(b):
        pltpu.make_async_copy(rbufs[b], acc_sh.at[dbufs[b]], ssem.at[b]).wait()

    H = CHUNK // 2

    def _start_gather_streams(b):
        pltpu.async_copy(x_hbm.at[sbufs[b].at[pl.ds(0, H)]],
                         rbufs[b].at[pl.ds(0, H)], gsem.at[b])
        pltpu.async_copy(x_hbm.at[sbufs[b].at[pl.ds(H, H)]],
                         rbufs[b].at[pl.ds(H, H)], gsem.at[b])

    def _wait_gather(b):
        pltpu.make_async_copy(x_hbm.at[sbufs[b].at[pl.ds(0, H)]],
                              rbufs[b].at[pl.ds(0, H)], gsem.at[b]).wait()
        pltpu.make_async_copy(x_hbm.at[sbufs[b].at[pl.ds(H, H)]],
                              rbufs[b].at[pl.ds(H, H)], gsem.at[b]).wait()

    _fetch_idx(0, 0)
    _fetch_idx(1, 1)
    _wait_idx(0, 0)
    _start_gather_streams(0)

    lax.fori_loop(0, WROWS, _zero_row, 0)
    lax.fori_loop(0, ROWS_PER_TILE // WROWS, _zero_acc, 0)
    plsc.subcore_barrier()

    def _group(g, _):
        for b in range(K):
            nx = (b + 1) % K
            nx2 = (b + 2) % K

            if b >= 2:
                _wait_scatter(nx2)
            else:
                @pl.when(g >= 1)
                def _():
                    _wait_scatter(nx2)

            def _start_gather(g=g, b=b, nx=nx):
                _wait_idx(g * K + b + 1, nx)
                _start_gather_streams(nx)

            if b < 3:
                _start_gather()
            else:
                @pl.when(g < N_GROUPS - 1)
                def _():
                    _start_gather()

            if b < 2:
                _fetch_idx(g * K + b + 2, nx2)
            else:
                @pl.when(g < N_GROUPS - 1)
                def _():
                    _fetch_idx(g * K + b + 2, nx2)

            _wait_gather(b)
            pltpu.async_copy(rbufs[b], acc_sh.at[dbufs[b]], ssem.at[b], add=True)
        return 0

    lax.fori_loop(0, N_GROUPS, _group, 0)

    _wait_scatter(2)
    _wait_scatter(3)

    tbase = e_base + N_PIPE * CHUNK
    pltpu.sync_copy(src_hbm.at[pl.ds(tbase, CHUNK)], sbufs[0])
    pltpu.sync_copy(dst_hbm.at[pl.ds(tbase, CHUNK)], dbufs[0])
    pltpu.async_copy(x_hbm.at[sbufs[0]], rbufs[0], gsem.at[0]).wait()
    pltpu.sync_copy(rbufs[0], acc_sh.at[dbufs[0]], add=True)

    plsc.subcore_barrier()

    for j in range(ROWS_PER_TILE // WROWS):
        r0 = sid * ROWS_PER_TILE + j * WROWS
        pltpu.async_copy(acc_sh.at[pl.ds(r0, WROWS)],
                         out_hbm.at[cid].at[pl.ds(r0, WROWS)], gsem.at[0])
    for j in range(ROWS_PER_TILE // WROWS):
        r0 = sid * ROWS_PER_TILE + j * WROWS
        pltpu.make_async_copy(acc_sh.at[pl.ds(r0, WROWS)],
                              out_hbm.at[cid].at[pl.ds(r0, WROWS)],
                              gsem.at[0]).wait()


def _combine_body(p_ref, o_ref):
    o_ref[...] = p_ref[0] + p_ref[1]


_combine = pl.pallas_call(
    _combine_body,
    grid=(8,),
    in_specs=[pl.BlockSpec((NC, 1280, D), lambda i: (0, i, 0))],
    out_specs=pl.BlockSpec((1280, D), lambda i: (i, 0)),
    out_shape=jax.ShapeDtypeStruct((N_PAD, D), jnp.float32),
)


@jax.jit
def kernel(x, edge_index):
    ei = edge_index.astype(jnp.int32)
    partials = _mp_sc(x, ei[0], ei[1])
    return _combine(partials)[:N_NODES]


# speedup vs baseline: 1.0432x; 1.0432x over previous
import functools

import jax
import jax.numpy as jnp
from jax import lax
from jax.experimental import pallas as pl
from jax.experimental.pallas import tpu as pltpu
from jax.experimental.pallas import tpu_sc as plsc

N_NODES = 10000
N_PAD = 10240
N_EDGES = 320000
D = 128
LANES = 16

NC = 2
NS = 16
NW = NC * NS
E_PER_W = N_EDGES // NW
CHUNK = 80
N_CHUNKS = E_PER_W // CHUNK
ROWS_PER_TILE = N_PAD // NS
WROWS = CHUNK

K = 4
N_PIPE = N_CHUNKS - 1
N_GROUPS = N_PIPE // K

_mesh = plsc.VectorSubcoreMesh(core_axis_name="c", subcore_axis_name="s")


@functools.partial(
    pl.kernel,
    out_type=jax.ShapeDtypeStruct((NC, N_PAD, D), jnp.float32),
    mesh=_mesh,
    scratch_types=[
        [pltpu.VMEM((CHUNK,), jnp.int32) for _ in range(K)],
        [pltpu.VMEM((CHUNK,), jnp.int32) for _ in range(K)],
        [pltpu.VMEM((CHUNK, D), jnp.float32) for _ in range(K)],
        pltpu.VMEM_SHARED((N_PAD, D), jnp.float32),
        pltpu.SemaphoreType.DMA((K,)),
        pltpu.SemaphoreType.DMA((K,)),
        pltpu.SemaphoreType.DMA((K,)),
    ],
)
def _mp_sc(x_hbm, src_hbm, dst_hbm, out_hbm, sbufs, dbufs, rbufs,
           acc_sh, isem, gsem, ssem):
    cid = lax.axis_index("c")
    sid = lax.axis_index("s")
    wid = sid * NC + cid
    e_base = wid * E_PER_W

    zeros = jnp.zeros((LANES,), jnp.float32)

    def _zero_row(i, _):
        for c in range(D // LANES):
            rbufs[3][i, pl.ds(c * LANES, LANES)] = zeros
        return 0

    def _zero_acc(j, _):
        pltpu.sync_copy(rbufs[3],
                        acc_sh.at[pl.ds(sid * ROWS_PER_TILE + j * WROWS, WROWS)])
        return 0

    def _fetch_idx(c, b):
        base = e_base + c * CHUNK
        pltpu.async_copy(src_hbm.at[pl.ds(base, CHUNK)], sbufs[b], isem.at[b])
        pltpu.async_copy(dst_hbm.at[pl.ds(base, CHUNK)], dbufs[b], isem.at[b])

    def _wait_idx(c, b):
        base = e_base + c * CHUNK
        pltpu.make_async_copy(src_hbm.at[pl.ds(base, CHUNK)], sbufs[b],
                              isem.at[b]).wait()
        pltpu.make_async_copy(dst_hbm.at[pl.ds(base, CHUNK)], dbufs[b],
                              isem.at[b]).wait()

    def _wait_scatter(b):
        pltpu.make_async_copy(rbufs[b], acc_sh.at[dbufs[b]], ssem.at[b]).wait()

    H = CHUNK // 2

    def _start_gather_streams(b):
        pltpu.async_copy(x_hbm.at[sbufs[b].at[pl.ds(0, H)]],
                         rbufs[b].at[pl.ds(0, H)], gsem.at[b])
        pltpu.async_copy(x_hbm.at[sbufs[b].at[pl.ds(H, H)]],
                         rbufs[b].at[pl.ds(H, H)], gsem.at[b])

    def _wait_gather(b):
        pltpu.make_async_copy(x_hbm.at[sbufs[b].at[pl.ds(0, H)]],
                              rbufs[b].at[pl.ds(0, H)], gsem.at[b]).wait()
        pltpu.make_async_copy(x_hbm.at[sbufs[b].at[pl.ds(H, H)]],
                              rbufs[b].at[pl.ds(H, H)], gsem.at[b]).wait()

    _fetch_idx(0, 0)
    _fetch_idx(1, 1)
    _wait_idx(0, 0)
    _start_gather_streams(0)

    lax.fori_loop(0, WROWS, _zero_row, 0)
    lax.fori_loop(0, ROWS_PER_TILE // WROWS, _zero_acc, 0)
    plsc.subcore_barrier()

    def _group(g, _):
        for b in range(K):
            nx = (b + 1) % K
            nx2 = (b + 2) % K

            if b >= 2:
                _wait_scatter(nx2)
            else:
                @pl.when(g >= 1)
                def _():
                    _wait_scatter(nx2)

            def _start_gather(g=g, b=b, nx=nx):
                _wait_idx(g * K + b + 1, nx)
                _start_gather_streams(nx)

            if b < 3:
                _start_gather()
            else:
                @pl.when(g < N_GROUPS - 1)
                def _():
                    _start_gather()

            if b < 2:
                _fetch_idx(g * K + b + 2, nx2)
            else:
                @pl.when(g < N_GROUPS - 1)
                def _():
                    _fetch_idx(g * K + b + 2, nx2)

            _wait_gather(b)
            pltpu.async_copy(rbufs[b], acc_sh.at[dbufs[b]], ssem.at[b], add=True)
        return 0

    lax.fori_loop(0, N_GROUPS, _group, 0)

    _wait_scatter(2)
    _wait_scatter(3)

    tbase = e_base + N_PIPE * CHUNK
    pltpu.sync_copy(src_hbm.at[pl.ds(tbase, CHUNK)], sbufs[0])
    pltpu.sync_copy(dst_hbm.at[pl.ds(tbase, CHUNK)], dbufs[0])
    pltpu.async_copy(x_hbm.at[sbufs[0]], rbufs[0], gsem.at[0]).wait()
    pltpu.sync_copy(rbufs[0], acc_sh.at[dbufs[0]], add=True)

    plsc.subcore_barrier()

    for j in range(ROWS_PER_TILE // WROWS):
        r0 = sid * ROWS_PER_TILE + j * WROWS
        pltpu.async_copy(acc_sh.at[pl.ds(r0, WROWS)],
                         out_hbm.at[cid].at[pl.ds(r0, WROWS)], gsem.at[0])
    for j in range(ROWS_PER_TILE // WROWS):
        r0 = sid * ROWS_PER_TILE + j * WROWS
        pltpu.make_async_copy(acc_sh.at[pl.ds(r0, WROWS)],
                              out_hbm.at[cid].at[pl.ds(r0, WROWS)],
                              gsem.at[0]).wait()


def _combine_body(p_ref, o_ref):
    o_ref[...] = p_ref[0] + p_ref[1]


_combine = pl.pallas_call(
    _combine_body,
    grid=(5,),
    in_specs=[pl.BlockSpec((NC, N_NODES // 5, D), lambda i: (0, i, 0))],
    out_specs=pl.BlockSpec((N_NODES // 5, D), lambda i: (i, 0)),
    out_shape=jax.ShapeDtypeStruct((N_NODES, D), jnp.float32),
)


@jax.jit
def kernel(x, edge_index):
    ei = edge_index.astype(jnp.int32)
    partials = _mp_sc(x, ei[0], ei[1])
    return _combine(partials)
